# baseline (speedup 1.0000x reference)
import jax
import jax.numpy as jnp
from jax.experimental import pallas as pl
from jax.experimental.pallas import tpu as pltpu

B, N, D = 8, 512, 128
NREF, L = 512, 16
TAU = 1.0
TEMP = 1.0


def _softmax_lanes(p):
    m = jnp.max(p, axis=1, keepdims=True)
    e = jnp.exp(p - m)
    s = jnp.sum(e, axis=1, keepdims=True)
    return e / s


def _sum11(x):
    return jnp.sum(jnp.sum(x, axis=1, keepdims=True), axis=0, keepdims=True)


def _prologue_kernel(ref_ref, wv_ref, wt_ref, rankr_ref, ssr_ref):
    refm = ref_ref[...]
    iota_col = jax.lax.broadcasted_iota(jnp.int32, (N, N), 0)
    iota_row = jax.lax.broadcasted_iota(jnp.int32, (N, N), 1)
    tri = iota_col < iota_row

    wv = wv_ref[...]
    row_norm = jnp.sqrt(jnp.sum(wv * wv, axis=1, keepdims=True))
    w = wv / row_norm
    wt = jnp.transpose(w)
    wt_ref[...] = wt
    rsl = jnp.dot(refm, wt, preferred_element_type=jnp.float32)
    reft = jnp.transpose(refm)
    rslt = jnp.dot(w, reft, preferred_element_type=jnp.float32)
    for l in range(L):
        r_col = rsl[:, l:l + 1]
        r_row = rslt[l:l + 1, :]
        c = jnp.where((r_col < r_row) | ((r_col == r_row) & tri),
                      1, 0).astype(jnp.int32)
        rank_row = jnp.sum(c, axis=0, keepdims=True)
        rank_col = (N - 1) - jnp.sum(c, axis=1, keepdims=True)
        rankr_ref[:, l:l + 1] = rank_col
        s = rank_row == ((N - 1) - iota_col)
        rsd_col = jnp.sum(jnp.where(s, r_row, 0.0), axis=1, keepdims=True)
        p = -((r_row - rsd_col) ** 2) / TAU
        ssr_ref[l] = _softmax_lanes(p)


def _main_kernel(x_ref, ref_ref, wt_ref, rankr_ref, ssr_ref,
                 emb_ref, dists_ref,
                 cost_sc, costb_sc, e_sc, ranks_sc, ed_sc, dl_sc):
    refm = ref_ref[...]
    iota_col = jax.lax.broadcasted_iota(jnp.int32, (N, N), 0)
    iota_row = jax.lax.broadcasted_iota(jnp.int32, (N, N), 1)
    tri = iota_col < iota_row
    b = pl.program_id(0)

    x_b = x_ref[0]
    xt = jnp.transpose(x_b)
    inner = jnp.dot(refm, xt, preferred_element_type=jnp.float32)
    r2 = jnp.sum(refm * refm, axis=1, keepdims=True)
    x2 = jnp.sum(xt * xt, axis=0, keepdims=True)
    cost_sc[...] = jnp.sqrt(jnp.maximum(r2 + x2 - 2.0 * inner, 1e-12))
    cost = cost_sc[...]
    costb_sc[...] = cost.astype(jnp.bfloat16)
    cost_bf = costb_sc[...]

    xsl = jnp.dot(x_b, wt_ref[...], preferred_element_type=jnp.float32)
    xslt = jnp.transpose(xsl)

    for l in range(L):
        x_col = xsl[:, l:l + 1]
        x_row = xslt[l:l + 1, :]
        c = jnp.where((x_col < x_row) | ((x_col == x_row) & tri),
                      1, 0).astype(jnp.int32)
        rank_row = jnp.sum(c, axis=0, keepdims=True)
        ranks_sc[l:l + 1, :] = rank_row
        m = rankr_ref[:, l:l + 1] == rank_row
        ed_sc[0:1, l:l + 1] = _sum11(jnp.where(m, cost, 0.0)) * (1.0 / N)
        s = rank_row == ((N - 1) - iota_col)
        xsd_col = jnp.sum(jnp.where(s, x_row, 0.0), axis=1, keepdims=True)
        p = -((x_row - xsd_col) ** 2) / TAU
        ss_x = _softmax_lanes(p).astype(jnp.bfloat16)
        g = jnp.dot(cost_bf, ss_x, preferred_element_type=jnp.float32)
        dl_sc[0:1, l:l + 1] = _sum11(ssr_ref[l] * g) * (1.0 / (N * B))

    dists_ref[pl.ds(b, 1), :] = dl_sc[0:1, :]

    ed_row = ed_sc[0:1, :]
    w_row = _softmax_lanes(-ed_row / TEMP)

    for l in range(L):
        m = rankr_ref[:, l:l + 1] == ranks_sc[l:l + 1, :]
        term = jnp.where(m, w_row[:, l:l + 1], 0.0)
        if l == 0:
            e_sc[...] = term
        else:
            e_sc[...] = e_sc[...] + term
    e = e_sc[...]
    bary = jnp.dot(e, x_b, preferred_element_type=jnp.float32)
    denom = jnp.sum(e, axis=1, keepdims=True) * (1.0 / N) + 1e-8
    emb_ref[0] = (bary * (1.0 / N)) / denom - refm


def kernel(X, reference, weight_v):
    wt, rankr, ssr = pl.pallas_call(
        _prologue_kernel,
        out_shape=[
            jax.ShapeDtypeStruct((D, L), jnp.float32),
            jax.ShapeDtypeStruct((NREF, L), jnp.int32),
            jax.ShapeDtypeStruct((L, NREF, N), jnp.float32),
        ],
    )(reference, weight_v)

    emb, dists = pl.pallas_call(
        _main_kernel,
        grid=(B,),
        in_specs=[
            pl.BlockSpec((1, N, D), lambda b: (b, 0, 0)),
            pl.BlockSpec((NREF, D), lambda b: (0, 0)),
            pl.BlockSpec((D, L), lambda b: (0, 0)),
            pl.BlockSpec((NREF, L), lambda b: (0, 0)),
            pl.BlockSpec((L, NREF, N), lambda b: (0, 0, 0)),
        ],
        out_specs=[
            pl.BlockSpec((1, NREF, D), lambda b: (b, 0, 0)),
            pl.BlockSpec((B, L), lambda b: (0, 0)),
        ],
        out_shape=[
            jax.ShapeDtypeStruct((B, NREF, D), jnp.float32),
            jax.ShapeDtypeStruct((B, L), jnp.float32),
        ],
        scratch_shapes=[
            pltpu.VMEM((NREF, N), jnp.float32),
            pltpu.VMEM((NREF, N), jnp.bfloat16),
            pltpu.VMEM((NREF, N), jnp.float32),
            pltpu.VMEM((L, N), jnp.int32),
            pltpu.VMEM((8, L), jnp.float32),
            pltpu.VMEM((8, L), jnp.float32),
        ],
    )(X, reference, wt, rankr, ssr)
    per_slice = jnp.sum(dists, axis=0)
    return emb, per_slice


# speedup vs baseline: 1.0939x; 1.0939x over previous
import jax
import jax.numpy as jnp
from jax.experimental import pallas as pl
from jax.experimental.pallas import tpu as pltpu

B, N, D = 8, 512, 128
NREF, L = 512, 16
TAU = 1.0
TEMP = 1.0


def _softmax_lanes(p):
    m = jnp.max(p, axis=1, keepdims=True)
    e = jnp.exp(p - m)
    s = jnp.sum(e, axis=1, keepdims=True)
    return e / s


def _sum11(x):
    return jnp.sum(jnp.sum(x, axis=1, keepdims=True), axis=0, keepdims=True)


def _fused_kernel(x_ref, ref_ref, wv_ref, emb_ref, dists_ref,
                  wt_sc, rankr_sc, ssr_sc, cost_sc, costb_sc, e_sc,
                  ranks_sc, ed_sc, dl_sc):
    b = pl.program_id(0)
    refm = ref_ref[...]
    iota_col = jax.lax.broadcasted_iota(jnp.int32, (N, N), 0)
    iota_row = jax.lax.broadcasted_iota(jnp.int32, (N, N), 1)
    tri = iota_col < iota_row

    @pl.when(b == 0)
    def _prologue():
        wv = wv_ref[...]
        row_norm = jnp.sqrt(jnp.sum(wv * wv, axis=1, keepdims=True))
        w = wv / row_norm
        wt_sc[...] = jnp.transpose(w)
        rsl = jnp.dot(refm, wt_sc[...], preferred_element_type=jnp.float32)
        reft = jnp.transpose(refm)
        rslt = jnp.dot(w, reft, preferred_element_type=jnp.float32)
        for l in range(L):
            r_col = rsl[:, l:l + 1]
            r_row = rslt[l:l + 1, :]
            c = jnp.where((r_col < r_row) | ((r_col == r_row) & tri),
                          1, 0).astype(jnp.int32)
            rank_row = jnp.sum(c, axis=0, keepdims=True)
            rank_col = (N - 1) - jnp.sum(c, axis=1, keepdims=True)
            rankr_sc[:, l:l + 1] = rank_col
            s = rank_row == ((N - 1) - iota_col)
            rsd_col = jnp.sum(jnp.where(s, r_row, 0.0), axis=1, keepdims=True)
            p = -((r_row - rsd_col) ** 2) / TAU
            ssr_sc[l] = _softmax_lanes(p)

    x_b = x_ref[0]
    xt = jnp.transpose(x_b)
    inner = jnp.dot(refm, xt, preferred_element_type=jnp.float32)
    r2 = jnp.sum(refm * refm, axis=1, keepdims=True)
    x2 = jnp.sum(xt * xt, axis=0, keepdims=True)
    cost_sc[...] = jnp.sqrt(jnp.maximum(r2 + x2 - 2.0 * inner, 1e-12))
    cost = cost_sc[...]
    costb_sc[...] = cost.astype(jnp.bfloat16)
    cost_bf = costb_sc[...]

    xsl = jnp.dot(x_b, wt_sc[...], preferred_element_type=jnp.float32)
    xslt = jnp.transpose(xsl)

    for l in range(L):
        x_col = xsl[:, l:l + 1]
        x_row = xslt[l:l + 1, :]
        c = jnp.where((x_col < x_row) | ((x_col == x_row) & tri),
                      1, 0).astype(jnp.int32)
        rank_row = jnp.sum(c, axis=0, keepdims=True)
        ranks_sc[l:l + 1, :] = rank_row
        m = rankr_sc[:, l:l + 1] == rank_row
        ed_sc[0:1, l:l + 1] = _sum11(jnp.where(m, cost, 0.0)) * (1.0 / N)
        s = rank_row == ((N - 1) - iota_col)
        xsd_col = jnp.sum(jnp.where(s, x_row, 0.0), axis=1, keepdims=True)
        d = x_row - xsd_col
        q = d * d
        qmin = jnp.min(q, axis=1, keepdims=True)
        e = jnp.exp(qmin - q)
        s_row = jnp.sum(e, axis=1, keepdims=True)
        ss_x = (e / s_row).astype(jnp.bfloat16)
        g = jnp.dot(cost_bf, ss_x, preferred_element_type=jnp.float32)
        dl_sc[0:1, l:l + 1] = _sum11(ssr_sc[l] * g) * (1.0 / (N * B))

    dists_ref[pl.ds(b, 1), :] = dl_sc[0:1, :]

    ed_row = ed_sc[0:1, :]
    w_row = _softmax_lanes(-ed_row / TEMP)

    for l in range(L):
        m = rankr_sc[:, l:l + 1] == ranks_sc[l:l + 1, :]
        term = jnp.where(m, w_row[:, l:l + 1], 0.0)
        if l == 0:
            e_sc[...] = term
        else:
            e_sc[...] = e_sc[...] + term
    e = e_sc[...]
    bary = jnp.dot(e, x_b, preferred_element_type=jnp.float32)
    denom = jnp.sum(e, axis=1, keepdims=True) * (1.0 / N) + 1e-8
    emb_ref[0] = (bary * (1.0 / N)) / denom - refm


def kernel(X, reference, weight_v):
    emb, dists = pl.pallas_call(
        _fused_kernel,
        grid=(B,),
        in_specs=[
            pl.BlockSpec((1, N, D), lambda b: (b, 0, 0)),
            pl.BlockSpec((NREF, D), lambda b: (0, 0)),
            pl.BlockSpec((L, D), lambda b: (0, 0)),
        ],
        out_specs=[
            pl.BlockSpec((1, NREF, D), lambda b: (b, 0, 0)),
            pl.BlockSpec((B, L), lambda b: (0, 0)),
        ],
        out_shape=[
            jax.ShapeDtypeStruct((B, NREF, D), jnp.float32),
            jax.ShapeDtypeStruct((B, L), jnp.float32),
        ],
        scratch_shapes=[
            pltpu.VMEM((D, L), jnp.float32),
            pltpu.VMEM((NREF, L), jnp.int32),
            pltpu.VMEM((L, NREF, N), jnp.float32),
            pltpu.VMEM((NREF, N), jnp.float32),
            pltpu.VMEM((NREF, N), jnp.bfloat16),
            pltpu.VMEM((NREF, N), jnp.float32),
            pltpu.VMEM((L, N), jnp.int32),
            pltpu.VMEM((8, L), jnp.float32),
            pltpu.VMEM((8, L), jnp.float32),
        ],
    )(X, reference, weight_v)
    per_slice = jnp.sum(dists, axis=0)
    return emb, per_slice


# speedup vs baseline: 1.1042x; 1.0094x over previous
import jax
import jax.numpy as jnp
from jax.experimental import pallas as pl
from jax.experimental.pallas import tpu as pltpu

B, N, D = 8, 512, 128
NREF, L = 512, 16
TAU = 1.0
TEMP = 1.0


def _softmax_lanes(p):
    m = jnp.max(p, axis=1, keepdims=True)
    e = jnp.exp(p - m)
    s = jnp.sum(e, axis=1, keepdims=True)
    return e / s


def _sum11(x):
    return jnp.sum(jnp.sum(x, axis=1, keepdims=True), axis=0, keepdims=True)


def _fused_kernel(x_ref, ref_ref, wv_ref, emb_ref, dists_ref,
                  wt_sc, rankr_sc, ssr_sc, cost_sc, costb_sc, e_sc,
                  ranks_sc, ed_sc, dl_sc):
    b = pl.program_id(0)
    refm = ref_ref[...]
    iota_col = jax.lax.broadcasted_iota(jnp.int32, (N, N), 0)
    iota_row = jax.lax.broadcasted_iota(jnp.int32, (N, N), 1)
    tri = iota_col < iota_row

    @pl.when(b == 0)
    def _prologue():
        wv = wv_ref[...]
        row_norm = jnp.sqrt(jnp.sum(wv * wv, axis=1, keepdims=True))
        w = wv / row_norm
        wt_sc[...] = jnp.transpose(w)
        rsl = jnp.dot(refm, wt_sc[...], preferred_element_type=jnp.float32)
        reft = jnp.transpose(refm)
        rslt = jnp.dot(w, reft, preferred_element_type=jnp.float32)
        for l in range(L):
            r_col = rsl[:, l:l + 1]
            r_row = rslt[l:l + 1, :]
            c = jnp.where((r_col < r_row) | ((r_col == r_row) & tri),
                          1, 0).astype(jnp.int32)
            rank_row = jnp.sum(c, axis=0, keepdims=True)
            rank_col = (N - 1) - jnp.sum(c, axis=1, keepdims=True)
            rankr_sc[:, l:l + 1] = rank_col
            s = rank_row == ((N - 1) - iota_col)
            rsd_col = jnp.sum(jnp.where(s, r_row, 0.0), axis=1, keepdims=True)
            d = r_row - rsd_col
            q = d * d
            qmin = jnp.min(q, axis=1, keepdims=True)
            e = jnp.exp(qmin - q)
            s_row = jnp.sum(e, axis=1, keepdims=True)
            ssr_sc[l] = e / s_row

    x_b = x_ref[0]
    xt = jnp.transpose(x_b)
    inner = jnp.dot(refm, xt, preferred_element_type=jnp.float32)
    r2 = jnp.sum(refm * refm, axis=1, keepdims=True)
    x2 = jnp.sum(xt * xt, axis=0, keepdims=True)
    cost_sc[...] = jnp.sqrt(jnp.maximum(r2 + x2 - 2.0 * inner, 1e-12))
    cost = cost_sc[...]
    costb_sc[...] = cost.astype(jnp.bfloat16)
    cost_bf = costb_sc[...]

    xsl = jnp.dot(x_b, wt_sc[...], preferred_element_type=jnp.float32)
    xslt = jnp.transpose(xsl)

    for l in range(L):
        x_col = xsl[:, l:l + 1]
        x_row = xslt[l:l + 1, :]
        c = jnp.where((x_col < x_row) | ((x_col == x_row) & tri),
                      1, 0).astype(jnp.int32)
        rank_row = jnp.sum(c, axis=0, keepdims=True)
        ranks_sc[l:l + 1, :] = rank_row
        m = rankr_sc[:, l:l + 1] == rank_row
        ed_sc[0:1, l:l + 1] = _sum11(jnp.where(m, cost, 0.0)) * (1.0 / N)
        s = rank_row == ((N - 1) - iota_col)
        xsd_col = jnp.sum(jnp.where(s, x_row, 0.0), axis=1, keepdims=True)
        d = x_row.astype(jnp.bfloat16) - xsd_col.astype(jnp.bfloat16)
        q = d * d
        qmin = jnp.min(q, axis=1, keepdims=True)
        e = jnp.exp(qmin - q).astype(jnp.float32)
        s_row = jnp.sum(e, axis=1, keepdims=True)
        ss_x = (e / s_row).astype(jnp.bfloat16)
        g = jnp.dot(cost_bf, ss_x, preferred_element_type=jnp.float32)
        dl_sc[0:1, l:l + 1] = _sum11(ssr_sc[l] * g) * (1.0 / (N * B))

    dists_ref[pl.ds(b, 1), :] = dl_sc[0:1, :]

    ed_row = ed_sc[0:1, :]
    w_row = _softmax_lanes(-ed_row / TEMP)

    for l in range(L):
        m = rankr_sc[:, l:l + 1] == ranks_sc[l:l + 1, :]
        term = jnp.where(m, w_row[:, l:l + 1], 0.0)
        if l == 0:
            e_sc[...] = term
        else:
            e_sc[...] = e_sc[...] + term
    e = e_sc[...]
    bary = jnp.dot(e, x_b, preferred_element_type=jnp.float32)
    denom = jnp.sum(e, axis=1, keepdims=True) * (1.0 / N) + 1e-8
    emb_ref[0] = (bary * (1.0 / N)) / denom - refm


def kernel(X, reference, weight_v):
    emb, dists = pl.pallas_call(
        _fused_kernel,
        grid=(B,),
        in_specs=[
            pl.BlockSpec((1, N, D), lambda b: (b, 0, 0)),
            pl.BlockSpec((NREF, D), lambda b: (0, 0)),
            pl.BlockSpec((L, D), lambda b: (0, 0)),
        ],
        out_specs=[
            pl.BlockSpec((1, NREF, D), lambda b: (b, 0, 0)),
            pl.BlockSpec((B, L), lambda b: (0, 0)),
        ],
        out_shape=[
            jax.ShapeDtypeStruct((B, NREF, D), jnp.float32),
            jax.ShapeDtypeStruct((B, L), jnp.float32),
        ],
        scratch_shapes=[
            pltpu.VMEM((D, L), jnp.float32),
            pltpu.VMEM((NREF, L), jnp.int32),
            pltpu.VMEM((L, NREF, N), jnp.float32),
            pltpu.VMEM((NREF, N), jnp.float32),
            pltpu.VMEM((NREF, N), jnp.bfloat16),
            pltpu.VMEM((NREF, N), jnp.float32),
            pltpu.VMEM((L, N), jnp.int32),
            pltpu.VMEM((8, L), jnp.float32),
            pltpu.VMEM((8, L), jnp.float32),
        ],
    )(X, reference, weight_v)
    per_slice = jnp.sum(dists, axis=0)
    return emb, per_slice


# speedup vs baseline: 1.3477x; 1.2205x over previous
import jax
import jax.numpy as jnp
from jax.experimental import pallas as pl
from jax.experimental.pallas import tpu as pltpu

B, N, D = 8, 512, 128
NREF, L = 512, 16
TAU = 1.0
TEMP = 1.0


def _softmax_lanes(p):
    m = jnp.max(p, axis=1, keepdims=True)
    e = jnp.exp(p - m)
    s = jnp.sum(e, axis=1, keepdims=True)
    return e / s


def _sum11(x):
    return jnp.sum(jnp.sum(x, axis=1, keepdims=True), axis=0, keepdims=True)


def _fused_kernel(x_ref, ref_ref, wv_ref, emb_ref, dists_ref,
                  wt_sc, rankr_sc, ssr_sc, cost_sc, costb_sc, e_sc,
                  ranks_sc, ed_sc, dl_sc, ssx_sc):
    b = pl.program_id(0)
    refm = ref_ref[...]
    iota_col = jax.lax.broadcasted_iota(jnp.int32, (N, N), 0)
    iota_row = jax.lax.broadcasted_iota(jnp.int32, (N, N), 1)
    tri = iota_col < iota_row

    @pl.when(b == 0)
    def _prologue():
        wv = wv_ref[...]
        row_norm = jnp.sqrt(jnp.sum(wv * wv, axis=1, keepdims=True))
        w = wv / row_norm
        wt_sc[...] = jnp.transpose(w)
        rsl = jnp.dot(refm, wt_sc[...], preferred_element_type=jnp.float32)
        reft = jnp.transpose(refm)
        rslt = jnp.dot(w, reft, preferred_element_type=jnp.float32)
        for l in range(L):
            r_col = rsl[:, l:l + 1]
            r_row = rslt[l:l + 1, :]
            c = jnp.where((r_col < r_row) | ((r_col == r_row) & tri),
                          1, 0).astype(jnp.int32)
            rank_row = jnp.sum(c, axis=0, keepdims=True)
            rank_col = (N - 1) - jnp.sum(c, axis=1, keepdims=True)
            rankr_sc[:, l:l + 1] = rank_col
            s = rank_row == ((N - 1) - iota_col)
            rsd_col = jnp.sum(jnp.where(s, r_row, 0.0), axis=1, keepdims=True)
            d = r_row - rsd_col
            q = d * d
            qmin = jnp.min(q, axis=1, keepdims=True)
            e = jnp.exp(qmin - q)
            s_row = jnp.sum(e, axis=1, keepdims=True)
            ssr_sc[l] = e / s_row

    x_b = x_ref[0]
    xt = jnp.transpose(x_b)
    inner = jnp.dot(refm, xt, preferred_element_type=jnp.float32)
    r2 = jnp.sum(refm * refm, axis=1, keepdims=True)
    x2 = jnp.sum(xt * xt, axis=0, keepdims=True)
    cost_sc[...] = jnp.sqrt(jnp.maximum(r2 + x2 - 2.0 * inner, 1e-12))
    cost = cost_sc[...]
    costb_sc[...] = cost.astype(jnp.bfloat16)
    cost_bf = costb_sc[...]

    xsl = jnp.dot(x_b, wt_sc[...], preferred_element_type=jnp.float32)
    xslt = jnp.transpose(xsl)

    for l in range(L):
        x_col = xsl[:, l:l + 1]
        x_row = xslt[l:l + 1, :]
        c = jnp.where((x_col < x_row) | ((x_col == x_row) & tri),
                      1, 0).astype(jnp.int32)
        ranks_sc[l:l + 1, :] = jnp.sum(c, axis=0, keepdims=True)

    for l in range(L):
        m = rankr_sc[:, l:l + 1] == ranks_sc[l:l + 1, :]
        ed_sc[0:1, l:l + 1] = _sum11(jnp.where(m, cost, 0.0)) * (1.0 / N)

    for l in range(L):
        x_row = xslt[l:l + 1, :]
        rank_row = ranks_sc[l:l + 1, :]
        s = rank_row == ((N - 1) - iota_col)
        xsd_col = jnp.sum(jnp.where(s, x_row, 0.0), axis=1, keepdims=True)
        d = x_row.astype(jnp.bfloat16) - xsd_col.astype(jnp.bfloat16)
        q = d * d
        qmin = jnp.min(q, axis=1, keepdims=True)
        e = jnp.exp(qmin - q).astype(jnp.float32)
        s_row = jnp.sum(e, axis=1, keepdims=True)
        ssx_sc[l] = (e / s_row).astype(jnp.bfloat16)

    for l in range(L):
        g = jnp.dot(cost_bf, ssx_sc[l], preferred_element_type=jnp.float32)
        dl_sc[0:1, l:l + 1] = _sum11(ssr_sc[l] * g) * (1.0 / (N * B))

    dists_ref[pl.ds(b, 1), :] = dl_sc[0:1, :]

    ed_row = ed_sc[0:1, :]
    w_row = _softmax_lanes(-ed_row / TEMP)

    for l in range(L):
        m = rankr_sc[:, l:l + 1] == ranks_sc[l:l + 1, :]
        term = jnp.where(m, w_row[:, l:l + 1], 0.0)
        if l == 0:
            e_sc[...] = term
        else:
            e_sc[...] = e_sc[...] + term
    e = e_sc[...]
    bary = jnp.dot(e, x_b, preferred_element_type=jnp.float32)
    denom = jnp.sum(e, axis=1, keepdims=True) * (1.0 / N) + 1e-8
    emb_ref[0] = (bary * (1.0 / N)) / denom - refm


def kernel(X, reference, weight_v):
    emb, dists = pl.pallas_call(
        _fused_kernel,
        grid=(B,),
        in_specs=[
            pl.BlockSpec((1, N, D), lambda b: (b, 0, 0)),
            pl.BlockSpec((NREF, D), lambda b: (0, 0)),
            pl.BlockSpec((L, D), lambda b: (0, 0)),
        ],
        out_specs=[
            pl.BlockSpec((1, NREF, D), lambda b: (b, 0, 0)),
            pl.BlockSpec((B, L), lambda b: (0, 0)),
        ],
        out_shape=[
            jax.ShapeDtypeStruct((B, NREF, D), jnp.float32),
            jax.ShapeDtypeStruct((B, L), jnp.float32),
        ],
        scratch_shapes=[
            pltpu.VMEM((D, L), jnp.float32),
            pltpu.VMEM((NREF, L), jnp.int32),
            pltpu.VMEM((L, NREF, N), jnp.float32),
            pltpu.VMEM((NREF, N), jnp.float32),
            pltpu.VMEM((NREF, N), jnp.bfloat16),
            pltpu.VMEM((NREF, N), jnp.float32),
            pltpu.VMEM((L, N), jnp.int32),
            pltpu.VMEM((8, L), jnp.float32),
            pltpu.VMEM((8, L), jnp.float32),
            pltpu.VMEM((L, NREF, N), jnp.bfloat16),
        ],
    )(X, reference, weight_v)
    per_slice = jnp.sum(dists, axis=0)
    return emb, per_slice


# speedup vs baseline: 1.3496x; 1.0014x over previous
import jax
import jax.numpy as jnp
from jax.experimental import pallas as pl
from jax.experimental.pallas import tpu as pltpu

B, N, D = 8, 512, 128
NREF, L = 512, 16
TAU = 1.0
TEMP = 1.0


def _softmax_lanes(p):
    m = jnp.max(p, axis=1, keepdims=True)
    e = jnp.exp(p - m)
    s = jnp.sum(e, axis=1, keepdims=True)
    return e / s


def _sum11(x):
    return jnp.sum(jnp.sum(x, axis=1, keepdims=True), axis=0, keepdims=True)


def _fused_kernel(x_ref, ref_ref, wv_ref, emb_ref, dists_ref,
                  wt_sc, rankr_sc, ssr_sc, cost_sc, costb_sc, e_sc,
                  ranks_sc, ed_sc, dl_sc, ssx_sc):
    b = pl.program_id(0)
    refm = ref_ref[...]
    iota_col = jax.lax.broadcasted_iota(jnp.int32, (N, N), 0)
    iota_row = jax.lax.broadcasted_iota(jnp.int32, (N, N), 1)
    tri = iota_col < iota_row

    @pl.when(b == 0)
    def _prologue():
        wv = wv_ref[...]
        row_norm = jnp.sqrt(jnp.sum(wv * wv, axis=1, keepdims=True))
        w = wv / row_norm
        wt_sc[...] = jnp.transpose(w)
        rsl = jnp.dot(refm, wt_sc[...], preferred_element_type=jnp.float32)
        reft = jnp.transpose(refm)
        rslt = jnp.dot(w, reft, preferred_element_type=jnp.float32)
        rank_rows = []
        for l in range(L):
            r_col = rsl[:, l:l + 1]
            r_row = rslt[l:l + 1, :]
            c = jnp.where((r_col < r_row) | ((r_col == r_row) & tri),
                          1, 0).astype(jnp.int32)
            rank_rows.append(jnp.sum(c, axis=0, keepdims=True))
            rankr_sc[:, l:l + 1] = (N - 1) - jnp.sum(c, axis=1, keepdims=True)
        for l in range(L):
            r_row = rslt[l:l + 1, :]
            s = rank_rows[l] == ((N - 1) - iota_col)
            rsd_col = jnp.sum(jnp.where(s, r_row, 0.0), axis=1, keepdims=True)
            d = r_row - rsd_col
            q = d * d
            qmin = jnp.min(q, axis=1, keepdims=True)
            e = jnp.exp(qmin - q)
            s_row = jnp.sum(e, axis=1, keepdims=True)
            ssr_sc[l] = e / s_row

    x_b = x_ref[0]
    xt = jnp.transpose(x_b)
    inner = jnp.dot(refm, xt, preferred_element_type=jnp.float32)
    r2 = jnp.sum(refm * refm, axis=1, keepdims=True)
    x2 = jnp.sum(xt * xt, axis=0, keepdims=True)
    cost_sc[...] = jnp.sqrt(jnp.maximum(r2 + x2 - 2.0 * inner, 1e-12))
    cost = cost_sc[...]
    costb_sc[...] = cost.astype(jnp.bfloat16)
    cost_bf = costb_sc[...]

    xsl = jnp.dot(x_b, wt_sc[...], preferred_element_type=jnp.float32)
    xslt = jnp.transpose(xsl)

    for l in range(L):
        x_col = xsl[:, l:l + 1]
        x_row = xslt[l:l + 1, :]
        c = jnp.where((x_col < x_row) | ((x_col == x_row) & tri),
                      1, 0).astype(jnp.int32)
        ranks_sc[l:l + 1, :] = jnp.sum(c, axis=0, keepdims=True)

    for l in range(L):
        m = rankr_sc[:, l:l + 1] == ranks_sc[l:l + 1, :]
        ed_sc[0:1, l:l + 1] = _sum11(jnp.where(m, cost, 0.0)) * (1.0 / N)

    for l in range(L):
        x_row = xslt[l:l + 1, :]
        rank_row = ranks_sc[l:l + 1, :]
        s = rank_row == ((N - 1) - iota_col)
        xsd_col = jnp.sum(jnp.where(s, x_row, 0.0), axis=1, keepdims=True)
        d = x_row.astype(jnp.bfloat16) - xsd_col.astype(jnp.bfloat16)
        q = d * d
        qmin = jnp.min(q, axis=1, keepdims=True)
        e = jnp.exp(qmin - q).astype(jnp.float32)
        s_row = jnp.sum(e, axis=1, keepdims=True)
        ssx_sc[l] = (e / s_row).astype(jnp.bfloat16)

    for l in range(L):
        g = jnp.dot(cost_bf, ssx_sc[l], preferred_element_type=jnp.float32)
        dl_sc[0:1, l:l + 1] = _sum11(ssr_sc[l] * g) * (1.0 / (N * B))

    dists_ref[pl.ds(b, 1), :] = dl_sc[0:1, :]

    ed_row = ed_sc[0:1, :]
    w_row = _softmax_lanes(-ed_row / TEMP)

    accs = [None, None, None, None]
    for l in range(L):
        m = rankr_sc[:, l:l + 1] == ranks_sc[l:l + 1, :]
        term = jnp.where(m, w_row[:, l:l + 1], 0.0)
        a = l % 4
        accs[a] = term if accs[a] is None else accs[a] + term
    e_sc[...] = (accs[0] + accs[1]) + (accs[2] + accs[3])
    e = e_sc[...]
    bary = jnp.dot(e, x_b, preferred_element_type=jnp.float32)
    denom = jnp.sum(e, axis=1, keepdims=True) * (1.0 / N) + 1e-8
    emb_ref[0] = (bary * (1.0 / N)) / denom - refm


def kernel(X, reference, weight_v):
    emb, dists = pl.pallas_call(
        _fused_kernel,
        grid=(B,),
        in_specs=[
            pl.BlockSpec((1, N, D), lambda b: (b, 0, 0)),
            pl.BlockSpec((NREF, D), lambda b: (0, 0)),
            pl.BlockSpec((L, D), lambda b: (0, 0)),
        ],
        out_specs=[
            pl.BlockSpec((1, NREF, D), lambda b: (b, 0, 0)),
            pl.BlockSpec((B, L), lambda b: (0, 0)),
        ],
        out_shape=[
            jax.ShapeDtypeStruct((B, NREF, D), jnp.float32),
            jax.ShapeDtypeStruct((B, L), jnp.float32),
        ],
        scratch_shapes=[
            pltpu.VMEM((D, L), jnp.float32),
            pltpu.VMEM((NREF, L), jnp.int32),
            pltpu.VMEM((L, NREF, N), jnp.float32),
            pltpu.VMEM((NREF, N), jnp.float32),
            pltpu.VMEM((NREF, N), jnp.bfloat16),
            pltpu.VMEM((NREF, N), jnp.float32),
            pltpu.VMEM((L, N), jnp.int32),
            pltpu.VMEM((8, L), jnp.float32),
            pltpu.VMEM((8, L), jnp.float32),
            pltpu.VMEM((L, NREF, N), jnp.bfloat16),
        ],
    )(X, reference, weight_v)
    per_slice = jnp.sum(dists, axis=0)
    return emb, per_slice


# speedup vs baseline: 1.3721x; 1.0167x over previous
import jax
import jax.numpy as jnp
from jax.experimental import pallas as pl
from jax.experimental.pallas import tpu as pltpu

B, N, D = 8, 512, 128
NREF, L = 512, 16
TAU = 1.0
TEMP = 1.0


def _softmax_lanes(p):
    m = jnp.max(p, axis=1, keepdims=True)
    e = jnp.exp(p - m)
    s = jnp.sum(e, axis=1, keepdims=True)
    return e / s


def _sum11(x):
    return jnp.sum(jnp.sum(x, axis=1, keepdims=True), axis=0, keepdims=True)


def _fused_kernel(x_ref, ref_ref, wv_ref, emb_ref, dists_ref,
                  wt_sc, rankr_sc, ssr_sc, cost_sc, costb_sc, e_sc,
                  ranks_sc, ed_sc, dl_sc, ssx_sc):
    b = pl.program_id(0)
    refm = ref_ref[...]
    iota_col = jax.lax.broadcasted_iota(jnp.int32, (N, N), 0)
    iota_row = jax.lax.broadcasted_iota(jnp.int32, (N, N), 1)
    tri = iota_col < iota_row

    @pl.when(b == 0)
    def _prologue():
        wv = wv_ref[...]
        row_norm = jnp.sqrt(jnp.sum(wv * wv, axis=1, keepdims=True))
        w = wv / row_norm
        wt_sc[...] = jnp.transpose(w)
        rsl = jnp.dot(refm, wt_sc[...], preferred_element_type=jnp.float32)
        reft = jnp.transpose(refm)
        rslt = jnp.dot(w, reft, preferred_element_type=jnp.float32)
        rank_rows = []
        for l in range(L):
            r_col = rsl[:, l:l + 1]
            r_row = rslt[l:l + 1, :]
            c = jnp.where((r_col < r_row) | ((r_col == r_row) & tri),
                          1, 0).astype(jnp.int32)
            rank_rows.append(jnp.sum(c, axis=0, keepdims=True))
            rankr_sc[:, l:l + 1] = (N - 1) - jnp.sum(c, axis=1, keepdims=True)
        for l in range(L):
            r_row = rslt[l:l + 1, :]
            s = rank_rows[l] == ((N - 1) - iota_col)
            rsd_col = jnp.sum(jnp.where(s, r_row, 0.0), axis=1, keepdims=True)
            d = r_row - rsd_col
            q = d * d
            qmin = jnp.min(q, axis=1, keepdims=True)
            e = jnp.exp(qmin - q)
            s_row = jnp.sum(e, axis=1, keepdims=True)
            ssr_sc[l] = e / s_row

    x_b = x_ref[0]
    xt = jnp.transpose(x_b)
    inner = jnp.dot(refm, xt, preferred_element_type=jnp.float32)
    r2 = jnp.sum(refm * refm, axis=1, keepdims=True)
    x2 = jnp.sum(xt * xt, axis=0, keepdims=True)
    cost_sc[...] = jnp.sqrt(jnp.maximum(r2 + x2 - 2.0 * inner, 1e-12))
    cost = cost_sc[...]
    costb_sc[...] = cost.astype(jnp.bfloat16)
    cost_bf = costb_sc[...]

    xsl = jnp.dot(x_b, wt_sc[...], preferred_element_type=jnp.float32)
    xslt = jnp.transpose(xsl)

    for l in range(L):
        x_col = xsl[:, l:l + 1]
        x_row = xslt[l:l + 1, :]
        c = jnp.where((x_col < x_row) | ((x_col == x_row) & tri),
                      1, 0).astype(jnp.int32)
        ranks_sc[l:l + 1, :] = jnp.sum(c, axis=0, keepdims=True)

    for l in range(L):
        m = rankr_sc[:, l:l + 1] == ranks_sc[l:l + 1, :]
        ed_sc[0:1, l:l + 1] = _sum11(jnp.where(m, cost, 0.0)) * (1.0 / N)

    def _softsort(l):
        x_row = xslt[l:l + 1, :]
        rank_row = ranks_sc[l:l + 1, :]
        s = rank_row == ((N - 1) - iota_col)
        xsd_col = jnp.sum(jnp.where(s, x_row, 0.0), axis=1, keepdims=True)
        d = x_row.astype(jnp.bfloat16) - xsd_col.astype(jnp.bfloat16)
        q = d * d
        qmin = jnp.min(q, axis=1, keepdims=True)
        e = jnp.exp(qmin - q).astype(jnp.float32)
        s_row = jnp.sum(e, axis=1, keepdims=True)
        ssx_sc[l] = (e / s_row).astype(jnp.bfloat16)

    def _dist(l):
        g = jnp.dot(cost_bf, ssx_sc[l], preferred_element_type=jnp.float32)
        dl_sc[0:1, l:l + 1] = _sum11(ssr_sc[l] * g) * (1.0 / (N * B))

    _softsort(0)
    for l in range(1, L):
        _softsort(l)
        _dist(l - 1)
    _dist(L - 1)

    dists_ref[pl.ds(b, 1), :] = dl_sc[0:1, :]

    ed_row = ed_sc[0:1, :]
    w_row = _softmax_lanes(-ed_row / TEMP)

    accs = [None, None, None, None]
    for l in range(L):
        m = rankr_sc[:, l:l + 1] == ranks_sc[l:l + 1, :]
        term = jnp.where(m, w_row[:, l:l + 1], 0.0)
        a = l % 4
        accs[a] = term if accs[a] is None else accs[a] + term
    e_sc[...] = (accs[0] + accs[1]) + (accs[2] + accs[3])
    e = e_sc[...]
    bary = jnp.dot(e, x_b, preferred_element_type=jnp.float32)
    denom = jnp.sum(e, axis=1, keepdims=True) * (1.0 / N) + 1e-8
    emb_ref[0] = (bary * (1.0 / N)) / denom - refm


def kernel(X, reference, weight_v):
    emb, dists = pl.pallas_call(
        _fused_kernel,
        grid=(B,),
        in_specs=[
            pl.BlockSpec((1, N, D), lambda b: (b, 0, 0)),
            pl.BlockSpec((NREF, D), lambda b: (0, 0)),
            pl.BlockSpec((L, D), lambda b: (0, 0)),
        ],
        out_specs=[
            pl.BlockSpec((1, NREF, D), lambda b: (b, 0, 0)),
            pl.BlockSpec((B, L), lambda b: (0, 0)),
        ],
        out_shape=[
            jax.ShapeDtypeStruct((B, NREF, D), jnp.float32),
            jax.ShapeDtypeStruct((B, L), jnp.float32),
        ],
        scratch_shapes=[
            pltpu.VMEM((D, L), jnp.float32),
            pltpu.VMEM((NREF, L), jnp.int32),
            pltpu.VMEM((L, NREF, N), jnp.float32),
            pltpu.VMEM((NREF, N), jnp.float32),
            pltpu.VMEM((NREF, N), jnp.bfloat16),
            pltpu.VMEM((NREF, N), jnp.float32),
            pltpu.VMEM((L, N), jnp.int32),
            pltpu.VMEM((8, L), jnp.float32),
            pltpu.VMEM((8, L), jnp.float32),
            pltpu.VMEM((L, NREF, N), jnp.bfloat16),
        ],
    )(X, reference, weight_v)
    per_slice = jnp.sum(dists, axis=0)
    return emb, per_slice
